# Initial kernel scaffold; baseline (speedup 1.0000x reference)
import jax
import jax.numpy as jnp
from jax.experimental import pallas as pl


def kernel(x, edge_index, batch, W1, b1, W2, b2, Wc1, bc1, Wc2, bc2):
    raise NotImplementedError("write your pallas kernel here")


# speedup vs baseline: 8.3355x; 8.3355x over previous
import functools

import jax
import jax.numpy as jnp
from jax import lax
from jax.experimental import pallas as pl
from jax.experimental.pallas import tpu as pltpu
from jax.experimental.pallas import tpu_sc as plsc

N = 10000
NP = 10240
E = 320000
EP = 327680
D = 128
G = 64
NCLS = 16
NC = 2
NS = 16
NW = NC * NS
EPW = EP // NW
CHUNK = 128
NCH = EPW // CHUNK
RPT = NP // NS
BLK = 1024
NBLK = NP // BLK

_mesh = plsc.VectorSubcoreMesh(
    core_axis_name="c", subcore_axis_name="s", num_cores=NC, num_subcores=NS)


@functools.partial(
    pl.kernel,
    out_type=jax.ShapeDtypeStruct((NC * NP,), jnp.float32),
    mesh=_mesh,
    scratch_types=[
        pltpu.VMEM((NCH, 2, CHUNK), jnp.int32),
        pltpu.VMEM((CHUNK,), jnp.float32),
        pltpu.VMEM_SHARED((NP,), jnp.float32),
    ],
)
def _deg_kernel(ec_hbm, zc_hbm, out_hbm, ibuf, ones_v, table):
    c = lax.axis_index("c")
    s = lax.axis_index("s")
    wid = s * NC + c
    for i in range(CHUNK // 16):
        ones_v[pl.ds(i * 16, 16)] = jnp.ones((16,), jnp.float32)
    pltpu.sync_copy(ec_hbm.at[wid], ibuf)
    pltpu.sync_copy(zc_hbm, table.at[pl.ds(s * RPT, RPT)])
    plsc.subcore_barrier()

    def body(i, carry):
        pltpu.sync_copy(ones_v, table.at[ibuf.at[i, 1]], add=True)
        return carry

    lax.fori_loop(0, NCH, body, 0)
    plsc.subcore_barrier()
    pltpu.sync_copy(table.at[pl.ds(s * RPT, RPT)],
                    out_hbm.at[pl.ds(c * NP + s * RPT, RPT)])


@functools.partial(
    pl.kernel,
    out_type=jax.ShapeDtypeStruct((NC * NP, D), jnp.float32),
    mesh=_mesh,
    scratch_types=[
        pltpu.VMEM((NCH, 2, CHUNK), jnp.int32),
        pltpu.VMEM((CHUNK, D), jnp.float32),
        pltpu.SemaphoreType.DMA,
        pltpu.VMEM_SHARED((NP, D), jnp.float32),
    ],
)
def _agg_kernel(y_hbm, ec_hbm, zr_hbm, out_hbm, ibuf, rows, sem, table):
    c = lax.axis_index("c")
    s = lax.axis_index("s")
    wid = s * NC + c
    pltpu.sync_copy(ec_hbm.at[wid], ibuf)
    pltpu.sync_copy(zr_hbm, table.at[pl.ds(s * RPT, RPT)])
    plsc.subcore_barrier()

    def body(i, carry):
        pltpu.async_copy(y_hbm.at[ibuf.at[i, 0]], rows, sem).wait()
        pltpu.sync_copy(rows, table.at[ibuf.at[i, 1]], add=True)
        return carry

    lax.fori_loop(0, NCH, body, 0)
    plsc.subcore_barrier()
    pltpu.sync_copy(table.at[pl.ds(s * RPT, RPT)],
                    out_hbm.at[pl.ds(c * NP + s * RPT, RPT)])


def _dinv(deg_ref):
    return lax.rsqrt(deg_ref[0, :] + deg_ref[1, :] + 1.0)


def _y1_body(x_ref, deg_ref, w_ref, y_ref):
    dinv = _dinv(deg_ref)
    y_ref[...] = jnp.dot(x_ref[...], w_ref[...],
                         preferred_element_type=jnp.float32) * dinv[:, None]


def _mid_body(agg_ref, y1_ref, deg_ref, w_ref, b_ref, y2_ref):
    dinv = _dinv(deg_ref)
    ssum = agg_ref[0] + agg_ref[1] + y1_ref[...]
    h = jnp.maximum(ssum * dinv[:, None] + b_ref[...], 0.0)
    y2_ref[...] = jnp.dot(h, w_ref[...],
                          preferred_element_type=jnp.float32) * dinv[:, None]


def _final_body(agg_ref, y2_ref, deg_ref, batch_ref, b2_ref, wc1_ref, bc1_ref,
                wc2_ref, bc2_ref, out_ref, acc, cnt):
    i = pl.program_id(0)

    @pl.when(i == 0)
    def _():
        acc[...] = jnp.zeros_like(acc)
        cnt[...] = jnp.zeros_like(cnt)

    dinv = _dinv(deg_ref)
    ssum = agg_ref[0] + agg_ref[1] + y2_ref[...]
    h = jnp.maximum(ssum * dinv[:, None] + b2_ref[...], 0.0)
    bb = batch_ref[0, 0, :]
    m = (lax.broadcasted_iota(jnp.int32, (G, BLK), 0)
         == bb[None, :]).astype(jnp.float32)
    acc[...] += jnp.dot(m, h, preferred_element_type=jnp.float32)
    cnt[...] += jnp.broadcast_to(jnp.sum(m, axis=1, keepdims=True), (G, D))

    @pl.when(i == NBLK - 1)
    def _():
        pooled = acc[...] / jnp.maximum(cnt[...], 1.0)
        z = jnp.maximum(jnp.dot(pooled, wc1_ref[...],
                                preferred_element_type=jnp.float32)
                        + bc1_ref[...], 0.0)
        out_ref[...] = jnp.dot(z, wc2_ref[...],
                               preferred_element_type=jnp.float32) + bc2_ref[...]


def kernel(x, edge_index, batch, W1, b1, W2, b2, Wc1, bc1, Wc2, bc2):
    row = edge_index[0].astype(jnp.int32)
    col = edge_index[1].astype(jnp.int32)
    pad_e = EP - E
    rowp = jnp.concatenate([row, jnp.zeros((pad_e,), jnp.int32)])
    colp = jnp.concatenate([col, jnp.full((pad_e,), N, jnp.int32)])
    ec = jnp.stack([rowp.reshape(NW, NCH, CHUNK),
                    colp.reshape(NW, NCH, CHUNK)], axis=2)
    xp = jnp.pad(x, ((0, NP - N), (0, 0)))
    batchp = jnp.concatenate(
        [batch.astype(jnp.int32), jnp.full((NP - N,), G, jnp.int32)]
    ).reshape(NBLK, 1, BLK)
    zc = jnp.zeros((RPT,), jnp.float32)
    zr = jnp.zeros((RPT, D), jnp.float32)

    deg2 = _deg_kernel(ec, zc).reshape(NC, NP)

    y1 = pl.pallas_call(
        _y1_body,
        grid=(NBLK,),
        in_specs=[
            pl.BlockSpec((BLK, D), lambda i: (i, 0)),
            pl.BlockSpec((NC, BLK), lambda i: (0, i)),
            pl.BlockSpec((D, D), lambda i: (0, 0)),
        ],
        out_specs=pl.BlockSpec((BLK, D), lambda i: (i, 0)),
        out_shape=jax.ShapeDtypeStruct((NP, D), jnp.float32),
    )(xp, deg2, W1)

    agg1 = _agg_kernel(y1, ec, zr).reshape(NC, NP, D)

    y2 = pl.pallas_call(
        _mid_body,
        grid=(NBLK,),
        in_specs=[
            pl.BlockSpec((NC, BLK, D), lambda i: (0, i, 0)),
            pl.BlockSpec((BLK, D), lambda i: (i, 0)),
            pl.BlockSpec((NC, BLK), lambda i: (0, i)),
            pl.BlockSpec((D, D), lambda i: (0, 0)),
            pl.BlockSpec((1, D), lambda i: (0, 0)),
        ],
        out_specs=pl.BlockSpec((BLK, D), lambda i: (i, 0)),
        out_shape=jax.ShapeDtypeStruct((NP, D), jnp.float32),
    )(agg1, y1, deg2, W2, b1.reshape(1, D))

    agg2 = _agg_kernel(y2, ec, zr).reshape(NC, NP, D)

    out = pl.pallas_call(
        _final_body,
        grid=(NBLK,),
        in_specs=[
            pl.BlockSpec((NC, BLK, D), lambda i: (0, i, 0)),
            pl.BlockSpec((BLK, D), lambda i: (i, 0)),
            pl.BlockSpec((NC, BLK), lambda i: (0, i)),
            pl.BlockSpec((1, 1, BLK), lambda i: (i, 0, 0)),
            pl.BlockSpec((1, D), lambda i: (0, 0)),
            pl.BlockSpec((D, D), lambda i: (0, 0)),
            pl.BlockSpec((1, D), lambda i: (0, 0)),
            pl.BlockSpec((D, NCLS), lambda i: (0, 0)),
            pl.BlockSpec((1, NCLS), lambda i: (0, 0)),
        ],
        out_specs=pl.BlockSpec((G, NCLS), lambda i: (0, 0)),
        out_shape=jax.ShapeDtypeStruct((G, NCLS), jnp.float32),
        scratch_shapes=[
            pltpu.VMEM((G, D), jnp.float32),
            pltpu.VMEM((G, D), jnp.float32),
        ],
    )(agg2, y2, deg2, batchp, b2.reshape(1, D), Wc1, bc1.reshape(1, D),
      Wc2, bc2.reshape(1, NCLS))

    return out


# speedup vs baseline: 9.1483x; 1.0975x over previous
import functools

import jax
import jax.numpy as jnp
from jax import lax
from jax.experimental import pallas as pl
from jax.experimental.pallas import tpu as pltpu
from jax.experimental.pallas import tpu_sc as plsc

N = 10000
NP = 10240
E = 320000
EP = 327680
D = 128
G = 64
NCLS = 16
NC = 2
NS = 16
NW = NC * NS
EPW = EP // NW
CHUNK = 128
NCH = EPW // CHUNK
RPT = NP // NS
BLK = 1024
NBLK = NP // BLK

_mesh = plsc.VectorSubcoreMesh(
    core_axis_name="c", subcore_axis_name="s", num_cores=NC, num_subcores=NS)


@functools.partial(
    pl.kernel,
    out_type=jax.ShapeDtypeStruct((NC * NP,), jnp.float32),
    mesh=_mesh,
    scratch_types=[
        pltpu.VMEM((NCH, 2, CHUNK), jnp.int32),
        pltpu.VMEM((CHUNK,), jnp.float32),
        pltpu.VMEM_SHARED((NP,), jnp.float32),
    ],
)
def _deg_kernel(ec_hbm, zc_hbm, out_hbm, ibuf, ones_v, table):
    c = lax.axis_index("c")
    s = lax.axis_index("s")
    wid = s * NC + c
    for i in range(CHUNK // 16):
        ones_v[pl.ds(i * 16, 16)] = jnp.ones((16,), jnp.float32)
    pltpu.sync_copy(ec_hbm.at[wid], ibuf)
    pltpu.sync_copy(zc_hbm, table.at[pl.ds(s * RPT, RPT)])
    plsc.subcore_barrier()

    def body(i, carry):
        pltpu.sync_copy(ones_v, table.at[ibuf.at[i, 1]], add=True)
        return carry

    lax.fori_loop(0, NCH, body, 0)
    plsc.subcore_barrier()
    pltpu.sync_copy(table.at[pl.ds(s * RPT, RPT)],
                    out_hbm.at[pl.ds(c * NP + s * RPT, RPT)])


NBUF = 2
NHALF = 2
HCH = NCH // NHALF


@functools.partial(
    pl.kernel,
    out_type=jax.ShapeDtypeStruct((NC * NP, D), jnp.float32),
    mesh=_mesh,
    scratch_types=[
        pltpu.VMEM((HCH, 2, CHUNK), jnp.int32),
        [pltpu.VMEM((CHUNK, D), jnp.float32) for _ in range(NBUF)],
        [pltpu.SemaphoreType.DMA for _ in range(NBUF)],
        pltpu.VMEM_SHARED((NP, D), jnp.float32),
    ],
)
def _agg_kernel(y_hbm, ec_hbm, zr_hbm, out_hbm, ibuf, rows, sems, table):
    c = lax.axis_index("c")
    s = lax.axis_index("s")
    wid = s * NC + c
    pltpu.sync_copy(zr_hbm, table.at[pl.ds(s * RPT, RPT)])
    plsc.subcore_barrier()

    for half in range(NHALF):
        pltpu.sync_copy(ec_hbm.at[wid, pl.ds(half * HCH, HCH)], ibuf)
        for k in range(NBUF):
            pltpu.async_copy(y_hbm.at[ibuf.at[k, 0]], rows[k], sems[k])

        def body(t, carry):
            i0 = t * NBUF
            for k in range(NBUF):
                i = i0 + k
                pltpu.make_async_copy(
                    y_hbm.at[ibuf.at[i, 0]], rows[k], sems[k]).wait()
                pltpu.sync_copy(rows[k], table.at[ibuf.at[i, 1]], add=True)

                @pl.when(t < HCH // NBUF - 1)
                def _():
                    pltpu.async_copy(
                        y_hbm.at[ibuf.at[i + NBUF, 0]], rows[k], sems[k])

            return carry

        lax.fori_loop(0, HCH // NBUF, body, 0)
    plsc.subcore_barrier()
    pltpu.sync_copy(table.at[pl.ds(s * RPT, RPT)],
                    out_hbm.at[pl.ds(c * NP + s * RPT, RPT)])


def _dinv(deg_ref):
    return lax.rsqrt(deg_ref[0, :] + deg_ref[1, :] + 1.0)


def _y1_body(x_ref, deg_ref, w_ref, y_ref):
    dinv = _dinv(deg_ref)
    y_ref[...] = jnp.dot(x_ref[...], w_ref[...],
                         preferred_element_type=jnp.float32) * dinv[:, None]


def _mid_body(agg_ref, y1_ref, deg_ref, w_ref, b_ref, y2_ref):
    dinv = _dinv(deg_ref)
    ssum = agg_ref[0] + agg_ref[1] + y1_ref[...]
    h = jnp.maximum(ssum * dinv[:, None] + b_ref[...], 0.0)
    y2_ref[...] = jnp.dot(h, w_ref[...],
                          preferred_element_type=jnp.float32) * dinv[:, None]


def _final_body(agg_ref, y2_ref, deg_ref, batch_ref, b2_ref, wc1_ref, bc1_ref,
                wc2_ref, bc2_ref, out_ref, acc, cnt):
    i = pl.program_id(0)

    @pl.when(i == 0)
    def _():
        acc[...] = jnp.zeros_like(acc)
        cnt[...] = jnp.zeros_like(cnt)

    dinv = _dinv(deg_ref)
    ssum = agg_ref[0] + agg_ref[1] + y2_ref[...]
    h = jnp.maximum(ssum * dinv[:, None] + b2_ref[...], 0.0)
    bb = batch_ref[0, 0, :]
    m = (lax.broadcasted_iota(jnp.int32, (G, BLK), 0)
         == bb[None, :]).astype(jnp.float32)
    acc[...] += jnp.dot(m, h, preferred_element_type=jnp.float32)
    cnt[...] += jnp.broadcast_to(jnp.sum(m, axis=1, keepdims=True), (G, D))

    @pl.when(i == NBLK - 1)
    def _():
        pooled = acc[...] / jnp.maximum(cnt[...], 1.0)
        z = jnp.maximum(jnp.dot(pooled, wc1_ref[...],
                                preferred_element_type=jnp.float32)
                        + bc1_ref[...], 0.0)
        out_ref[...] = jnp.dot(z, wc2_ref[...],
                               preferred_element_type=jnp.float32) + bc2_ref[...]


def kernel(x, edge_index, batch, W1, b1, W2, b2, Wc1, bc1, Wc2, bc2):
    row = edge_index[0].astype(jnp.int32)
    col = edge_index[1].astype(jnp.int32)
    pad_e = EP - E
    rowp = jnp.concatenate([row, jnp.zeros((pad_e,), jnp.int32)])
    colp = jnp.concatenate([col, jnp.full((pad_e,), N, jnp.int32)])
    ec = jnp.stack([rowp.reshape(NW, NCH, CHUNK),
                    colp.reshape(NW, NCH, CHUNK)], axis=2)
    xp = jnp.pad(x, ((0, NP - N), (0, 0)))
    batchp = jnp.concatenate(
        [batch.astype(jnp.int32), jnp.full((NP - N,), G, jnp.int32)]
    ).reshape(NBLK, 1, BLK)
    zc = jnp.zeros((RPT,), jnp.float32)
    zr = jnp.zeros((RPT, D), jnp.float32)

    deg2 = _deg_kernel(ec, zc).reshape(NC, NP)

    y1 = pl.pallas_call(
        _y1_body,
        grid=(NBLK,),
        in_specs=[
            pl.BlockSpec((BLK, D), lambda i: (i, 0)),
            pl.BlockSpec((NC, BLK), lambda i: (0, i)),
            pl.BlockSpec((D, D), lambda i: (0, 0)),
        ],
        out_specs=pl.BlockSpec((BLK, D), lambda i: (i, 0)),
        out_shape=jax.ShapeDtypeStruct((NP, D), jnp.float32),
    )(xp, deg2, W1)

    agg1 = _agg_kernel(y1, ec, zr).reshape(NC, NP, D)

    y2 = pl.pallas_call(
        _mid_body,
        grid=(NBLK,),
        in_specs=[
            pl.BlockSpec((NC, BLK, D), lambda i: (0, i, 0)),
            pl.BlockSpec((BLK, D), lambda i: (i, 0)),
            pl.BlockSpec((NC, BLK), lambda i: (0, i)),
            pl.BlockSpec((D, D), lambda i: (0, 0)),
            pl.BlockSpec((1, D), lambda i: (0, 0)),
        ],
        out_specs=pl.BlockSpec((BLK, D), lambda i: (i, 0)),
        out_shape=jax.ShapeDtypeStruct((NP, D), jnp.float32),
    )(agg1, y1, deg2, W2, b1.reshape(1, D))

    agg2 = _agg_kernel(y2, ec, zr).reshape(NC, NP, D)

    out = pl.pallas_call(
        _final_body,
        grid=(NBLK,),
        in_specs=[
            pl.BlockSpec((NC, BLK, D), lambda i: (0, i, 0)),
            pl.BlockSpec((BLK, D), lambda i: (i, 0)),
            pl.BlockSpec((NC, BLK), lambda i: (0, i)),
            pl.BlockSpec((1, 1, BLK), lambda i: (i, 0, 0)),
            pl.BlockSpec((1, D), lambda i: (0, 0)),
            pl.BlockSpec((D, D), lambda i: (0, 0)),
            pl.BlockSpec((1, D), lambda i: (0, 0)),
            pl.BlockSpec((D, NCLS), lambda i: (0, 0)),
            pl.BlockSpec((1, NCLS), lambda i: (0, 0)),
        ],
        out_specs=pl.BlockSpec((G, NCLS), lambda i: (0, 0)),
        out_shape=jax.ShapeDtypeStruct((G, NCLS), jnp.float32),
        scratch_shapes=[
            pltpu.VMEM((G, D), jnp.float32),
            pltpu.VMEM((G, D), jnp.float32),
        ],
    )(agg2, y2, deg2, batchp, b2.reshape(1, D), Wc1, bc1.reshape(1, D),
      Wc2, bc2.reshape(1, NCLS))

    return out
